# baseline (device time: 105065 ns/iter reference)
import jax
import jax.numpy as jnp
from jax import lax
from jax.experimental import pallas as pl
from jax.experimental.pallas import tpu as pltpu

N_DEV = 16
B, SQ, D = 2, 256, 768
DH = 64
HQ_LOC = 8
HKV_LOC = 2
ROWS = B * SQ
CHUNK = ROWS // N_DEV


def kernel(x, Wq, Wo, Wk, Wv):
    x2 = x.reshape(ROWS, D)
    Wk3 = Wk.reshape(D, N_DEV, HKV_LOC * DH)
    Wv3 = Wv.reshape(D, N_DEV, HKV_LOC * DH)

    def body(x_ref, wq_ref, wo_ref, wk_ref, wv_ref, out_ref,
             attn_ref, acc_ref, comm_ref,
             rs_send, rs_recv, ag_send, ag_recv):
        my_i = lax.axis_index("i")
        right = (my_i + 1) % N_DEV
        left = (my_i + N_DEV - 1) % N_DEV

        xv = x_ref[...]
        q = jnp.dot(xv, wq_ref[...], preferred_element_type=jnp.float32)
        wk_loc = wk_ref[:, my_i, :]
        wv_loc = wv_ref[:, my_i, :]
        k = jnp.dot(xv, wk_loc, preferred_element_type=jnp.float32)
        v = jnp.dot(xv, wv_loc, preferred_element_type=jnp.float32)

        for b in range(B):
            r0, r1 = b * SQ, (b + 1) * SQ
            for h in range(HQ_LOC):
                g = h // 4
                qh = q[r0:r1, h * DH:(h + 1) * DH]
                kh = k[r0:r1, g * DH:(g + 1) * DH]
                vh = v[r0:r1, g * DH:(g + 1) * DH]
                s = lax.dot_general(
                    qh, kh, (((1,), (1,)), ((), ())),
                    preferred_element_type=jnp.float32) * 0.125
                m = jnp.max(s, axis=1, keepdims=True)
                p = jnp.exp(s - m)
                l = jnp.sum(p, axis=1, keepdims=True)
                o = jnp.dot(p, vh, preferred_element_type=jnp.float32) / l
                attn_ref[r0:r1, h * DH:(h + 1) * DH] = o

        acc_ref[...] = jnp.dot(attn_ref[...], wo_ref[...],
                               preferred_element_type=jnp.float32)

        barrier = pltpu.get_barrier_semaphore()
        for nbr in (left, right):
            pl.semaphore_signal(barrier, inc=1, device_id=(nbr,),
                                device_id_type=pl.DeviceIdType.MESH)
        pl.semaphore_wait(barrier, 2)

        for s in range(N_DEV - 1):
            send_c = (my_i + (N_DEV - s)) % N_DEV
            recv_c = (my_i + (N_DEV - 1 - s)) % N_DEV
            rdma = pltpu.make_async_remote_copy(
                src_ref=acc_ref.at[pl.ds(send_c * CHUNK, CHUNK), :],
                dst_ref=comm_ref.at[s],
                send_sem=rs_send.at[s],
                recv_sem=rs_recv.at[s],
                device_id=(right,),
                device_id_type=pl.DeviceIdType.MESH,
            )
            rdma.start()
            rdma.wait()
            acc_ref[pl.ds(recv_c * CHUNK, CHUNK), :] = (
                acc_ref[pl.ds(recv_c * CHUNK, CHUNK), :] + comm_ref[s])

        own_c = (my_i + 1) % N_DEV
        out_ref[pl.ds(own_c * CHUNK, CHUNK), :] = (
            acc_ref[pl.ds(own_c * CHUNK, CHUNK), :])

        for t in range(N_DEV - 1):
            send_c = (my_i + (N_DEV + 1 - t)) % N_DEV
            rdma = pltpu.make_async_remote_copy(
                src_ref=out_ref.at[pl.ds(send_c * CHUNK, CHUNK), :],
                dst_ref=out_ref.at[pl.ds(send_c * CHUNK, CHUNK), :],
                send_sem=ag_send.at[t],
                recv_sem=ag_recv.at[t],
                device_id=(right,),
                device_id_type=pl.DeviceIdType.MESH,
            )
            rdma.start()
            rdma.wait()

    out = pl.pallas_call(
        body,
        out_shape=jax.ShapeDtypeStruct((ROWS, D), jnp.float32),
        in_specs=[pl.BlockSpec(memory_space=pltpu.VMEM)] * 5,
        out_specs=pl.BlockSpec(memory_space=pltpu.VMEM),
        scratch_shapes=[
            pltpu.VMEM((ROWS, HQ_LOC * DH), jnp.float32),
            pltpu.VMEM((ROWS, D), jnp.float32),
            pltpu.VMEM((N_DEV - 1, CHUNK, D), jnp.float32),
            pltpu.SemaphoreType.DMA((N_DEV - 1,)),
            pltpu.SemaphoreType.DMA((N_DEV - 1,)),
            pltpu.SemaphoreType.DMA((N_DEV - 1,)),
            pltpu.SemaphoreType.DMA((N_DEV - 1,)),
        ],
        compiler_params=pltpu.CompilerParams(collective_id=0),
    )(x2, Wq, Wo, Wk3, Wv3)
    return out.reshape(B, SQ, D)


# device time: 79980 ns/iter; 1.3136x vs baseline; 1.3136x over previous
import jax
import jax.numpy as jnp
from jax import lax
from jax.experimental import pallas as pl
from jax.experimental.pallas import tpu as pltpu

N_DEV = 16
B, SQ, D = 2, 256, 768
DH = 64
HQ_LOC = 8
HKV_LOC = 2
ROWS = B * SQ
CHUNK = ROWS // N_DEV


def kernel(x, Wq, Wo, Wk, Wv):
    x2 = x.reshape(ROWS, D)
    Wk3 = Wk.reshape(D, N_DEV, HKV_LOC * DH)
    Wv3 = Wv.reshape(D, N_DEV, HKV_LOC * DH)

    def body(x_ref, wq_ref, wo_ref, wk_ref, wv_ref, out_ref,
             attn_ref, acc_ref, comm_ref,
             rs_send, rs_recv, ag_send, ag_recv):
        my_i = lax.axis_index("i")
        right = (my_i + 1) % N_DEV
        left = (my_i + N_DEV - 1) % N_DEV

        xv = x_ref[...]
        q = jnp.dot(xv, wq_ref[...], preferred_element_type=jnp.float32)
        wk_loc = wk_ref[:, my_i, :]
        wv_loc = wv_ref[:, my_i, :]
        k = jnp.dot(xv, wk_loc, preferred_element_type=jnp.float32)
        v = jnp.dot(xv, wv_loc, preferred_element_type=jnp.float32)

        for b in range(B):
            r0, r1 = b * SQ, (b + 1) * SQ
            for h in range(HQ_LOC):
                g = h // 4
                qh = q[r0:r1, h * DH:(h + 1) * DH]
                kh = k[r0:r1, g * DH:(g + 1) * DH]
                vh = v[r0:r1, g * DH:(g + 1) * DH]
                s = lax.dot_general(
                    qh, kh, (((1,), (1,)), ((), ())),
                    preferred_element_type=jnp.float32) * 0.125
                m = jnp.max(s, axis=1, keepdims=True)
                p = jnp.exp(s - m)
                l = jnp.sum(p, axis=1, keepdims=True)
                o = jnp.dot(p, vh, preferred_element_type=jnp.float32) / l
                attn_ref[r0:r1, h * DH:(h + 1) * DH] = o

        acc_ref[...] = jnp.dot(attn_ref[...], wo_ref[...],
                               preferred_element_type=jnp.float32)

        barrier = pltpu.get_barrier_semaphore()
        for lg in range(4):
            pl.semaphore_signal(barrier, inc=1, device_id=(my_i ^ (1 << lg),),
                                device_id_type=pl.DeviceIdType.MESH)
        pl.semaphore_wait(barrier, 4)

        lo = 0
        off = 0
        for r in range(4):
            half = 8 >> r
            rows = half * CHUNK
            bit = (my_i >> (3 - r)) & 1
            send_lo = (lo + (1 - bit) * half) * CHUNK
            keep_lo = (lo + bit * half) * CHUNK
            partner = my_i ^ half
            rdma = pltpu.make_async_remote_copy(
                src_ref=acc_ref.at[pl.ds(send_lo, rows), :],
                dst_ref=comm_ref.at[pl.ds(off, rows), :],
                send_sem=rs_send.at[r],
                recv_sem=rs_recv.at[r],
                device_id=(partner,),
                device_id_type=pl.DeviceIdType.MESH,
            )
            rdma.start()
            rdma.wait()
            acc_ref[pl.ds(keep_lo, rows), :] = (
                acc_ref[pl.ds(keep_lo, rows), :]
                + comm_ref[pl.ds(off, rows), :])
            lo = lo + bit * half
            off += rows

        my_rows = my_i * CHUNK
        out_ref[pl.ds(my_rows, CHUNK), :] = acc_ref[pl.ds(my_rows, CHUNK), :]

        for j in range(4):
            rows = (1 << j) * CHUNK
            send_lo = ((my_i >> j) << j) * CHUNK
            partner = my_i ^ (1 << j)
            rdma = pltpu.make_async_remote_copy(
                src_ref=out_ref.at[pl.ds(send_lo, rows), :],
                dst_ref=out_ref.at[pl.ds(send_lo, rows), :],
                send_sem=ag_send.at[j],
                recv_sem=ag_recv.at[j],
                device_id=(partner,),
                device_id_type=pl.DeviceIdType.MESH,
            )
            rdma.start()
            rdma.wait()

    out = pl.pallas_call(
        body,
        out_shape=jax.ShapeDtypeStruct((ROWS, D), jnp.float32),
        in_specs=[pl.BlockSpec(memory_space=pltpu.VMEM)] * 5,
        out_specs=pl.BlockSpec(memory_space=pltpu.VMEM),
        scratch_shapes=[
            pltpu.VMEM((ROWS, HQ_LOC * DH), jnp.float32),
            pltpu.VMEM((ROWS, D), jnp.float32),
            pltpu.VMEM((480, D), jnp.float32),
            pltpu.SemaphoreType.DMA((4,)),
            pltpu.SemaphoreType.DMA((4,)),
            pltpu.SemaphoreType.DMA((4,)),
            pltpu.SemaphoreType.DMA((4,)),
        ],
        compiler_params=pltpu.CompilerParams(collective_id=0),
    )(x2, Wq, Wo, Wk3, Wv3)
    return out.reshape(B, SQ, D)


# device time: 63939 ns/iter; 1.6432x vs baseline; 1.2509x over previous
import jax
import jax.numpy as jnp
from jax import lax
from jax.experimental import pallas as pl
from jax.experimental.pallas import tpu as pltpu

N_DEV = 16
B, SQ, D = 2, 256, 768
DH = 64
HQ_LOC = 8
HKV_LOC = 2
ROWS = B * SQ
CHUNK = ROWS // N_DEV


def kernel(x, Wq, Wo, Wk, Wv):
    x2 = x.reshape(ROWS, D)
    Wk3 = Wk.reshape(D, N_DEV, HKV_LOC * DH)
    Wv3 = Wv.reshape(D, N_DEV, HKV_LOC * DH)

    def body(x_ref, wq_ref, wo_ref, wk_ref, wv_ref, out_ref,
             attn_ref, acc_ref, comm_ref,
             rs_send, rs_recv, ag_send, ag_recv):
        my_i = lax.axis_index("i")

        barrier = pltpu.get_barrier_semaphore()
        for lg in range(4):
            pl.semaphore_signal(barrier, inc=1, device_id=(my_i ^ (1 << lg),),
                                device_id_type=pl.DeviceIdType.MESH)
        pl.semaphore_wait(barrier, 4)

        wq = wq_ref[...]
        wo = wo_ref[...]
        wk_loc = wk_ref[:, my_i, :]
        wv_loc = wv_ref[:, my_i, :]

        def compute_batch(b):
            r0, r1 = b * SQ, (b + 1) * SQ
            xb = x_ref[r0:r1, :]
            q = jnp.dot(xb, wq, preferred_element_type=jnp.float32)
            k = jnp.dot(xb, wk_loc, preferred_element_type=jnp.float32)
            v = jnp.dot(xb, wv_loc, preferred_element_type=jnp.float32)
            for h in range(HQ_LOC):
                g = h // 4
                qh = q[:, h * DH:(h + 1) * DH]
                kh = k[:, g * DH:(g + 1) * DH]
                vh = v[:, g * DH:(g + 1) * DH]
                s = lax.dot_general(
                    qh, kh, (((1,), (1,)), ((), ())),
                    preferred_element_type=jnp.float32) * 0.125
                m = jnp.max(s, axis=1, keepdims=True)
                p = jnp.exp(s - m)
                l = jnp.sum(p, axis=1, keepdims=True)
                o = jnp.dot(p, vh, preferred_element_type=jnp.float32) / l
                attn_ref[r0:r1, h * DH:(h + 1) * DH] = o
            acc_ref[r0:r1, :] = jnp.dot(
                attn_ref[r0:r1, :], wo, preferred_element_type=jnp.float32)

        bit0 = my_i & 1

        @pl.when(bit0 == 0)
        def _():
            compute_batch(1)

        @pl.when(bit0 == 1)
        def _():
            compute_batch(0)

        lo = 0
        off = 0
        rdma0 = None
        for r in range(4):
            size = 256 >> r
            bit = (my_i >> r) & 1
            send_lo = lo + (1 - bit) * size
            keep_lo = lo + bit * size
            partner = my_i ^ (1 << r)
            rdma = pltpu.make_async_remote_copy(
                src_ref=acc_ref.at[pl.ds(send_lo, size), :],
                dst_ref=comm_ref.at[pl.ds(off, size), :],
                send_sem=rs_send.at[r],
                recv_sem=rs_recv.at[r],
                device_id=(partner,),
                device_id_type=pl.DeviceIdType.MESH,
            )
            rdma.start()
            if r == 0:
                @pl.when(bit0 == 0)
                def _():
                    compute_batch(0)

                @pl.when(bit0 == 1)
                def _():
                    compute_batch(1)

            rdma.wait()
            acc_ref[pl.ds(keep_lo, size), :] = (
                acc_ref[pl.ds(keep_lo, size), :]
                + comm_ref[pl.ds(off, size), :])
            lo = keep_lo
            off += size

        out_ref[pl.ds(lo, CHUNK), :] = acc_ref[pl.ds(lo, CHUNK), :]
        rev = lo // CHUNK

        for j in range(4):
            size = CHUNK << j
            send_lo = ((rev >> j) << j) * CHUNK
            partner = my_i ^ (8 >> j)
            rdma = pltpu.make_async_remote_copy(
                src_ref=out_ref.at[pl.ds(send_lo, size), :],
                dst_ref=out_ref.at[pl.ds(send_lo, size), :],
                send_sem=ag_send.at[j],
                recv_sem=ag_recv.at[j],
                device_id=(partner,),
                device_id_type=pl.DeviceIdType.MESH,
            )
            rdma.start()
            rdma.wait()

    out = pl.pallas_call(
        body,
        out_shape=jax.ShapeDtypeStruct((ROWS, D), jnp.float32),
        in_specs=[pl.BlockSpec(memory_space=pltpu.VMEM)] * 5,
        out_specs=pl.BlockSpec(memory_space=pltpu.VMEM),
        scratch_shapes=[
            pltpu.VMEM((ROWS, HQ_LOC * DH), jnp.float32),
            pltpu.VMEM((ROWS, D), jnp.float32),
            pltpu.VMEM((480, D), jnp.float32),
            pltpu.SemaphoreType.DMA((4,)),
            pltpu.SemaphoreType.DMA((4,)),
            pltpu.SemaphoreType.DMA((4,)),
            pltpu.SemaphoreType.DMA((4,)),
        ],
        compiler_params=pltpu.CompilerParams(collective_id=0),
    )(x2, Wq, Wo, Wk3, Wv3)
    return out.reshape(B, SQ, D)


# device time: 47730 ns/iter; 2.2012x vs baseline; 1.3396x over previous
import jax
import jax.numpy as jnp
from jax import lax
from jax.experimental import pallas as pl
from jax.experimental.pallas import tpu as pltpu

N_DEV = 16
B, SQ, D = 2, 256, 768
DH = 64
HQ_LOC = 8
HKV_LOC = 2
ROWS = B * SQ
CHUNK = ROWS // N_DEV


def kernel(x, Wq, Wo, Wk, Wv):
    x2 = x.reshape(ROWS, D)
    Wk3 = Wk.reshape(D, N_DEV, HKV_LOC * DH)
    Wv3 = Wv.reshape(D, N_DEV, HKV_LOC * DH)

    def body(x_ref, wq_ref, wo_ref, wk_ref, wv_ref, out_ref,
             attn_ref, acc_ref, sstage_ref, rstage_ref, g_ref,
             rs_send, rs_recv, ag_send, ag_recv):
        my_i = lax.axis_index("i")

        barrier = pltpu.get_barrier_semaphore()
        for lg in range(4):
            pl.semaphore_signal(barrier, inc=1, device_id=(my_i ^ (1 << lg),),
                                device_id_type=pl.DeviceIdType.MESH)
        pl.semaphore_wait(barrier, 4)

        wq = wq_ref[...]
        wo = wo_ref[...]
        wk_loc = wk_ref[:, my_i, :]
        wv_loc = wv_ref[:, my_i, :]

        def compute_batch(b):
            r0, r1 = b * SQ, (b + 1) * SQ
            xb = x_ref[r0:r1, :]
            q = jnp.dot(xb, wq, preferred_element_type=jnp.float32)
            k = jnp.dot(xb, wk_loc, preferred_element_type=jnp.float32)
            v = jnp.dot(xb, wv_loc, preferred_element_type=jnp.float32)
            for h in range(HQ_LOC):
                g = h // 4
                qh = q[:, h * DH:(h + 1) * DH]
                kh = k[:, g * DH:(g + 1) * DH]
                vh = v[:, g * DH:(g + 1) * DH]
                s = lax.dot_general(
                    qh, kh, (((1,), (1,)), ((), ())),
                    preferred_element_type=jnp.float32) * 0.125
                m = jnp.max(s, axis=1, keepdims=True)
                p = jnp.exp(s - m)
                l = jnp.sum(p, axis=1, keepdims=True)
                o = jnp.dot(p, vh, preferred_element_type=jnp.float32) / l
                attn_ref[r0:r1, h * DH:(h + 1) * DH] = o
            acc_ref[r0:r1, :] = jnp.dot(
                attn_ref[r0:r1, :], wo, preferred_element_type=jnp.float32)

        bit0 = my_i & 1

        @pl.when(bit0 == 0)
        def _():
            compute_batch(1)

        @pl.when(bit0 == 1)
        def _():
            compute_batch(0)

        lo = 0
        off = 0
        for r in range(4):
            size = 256 >> r
            bit = (my_i >> r) & 1
            send_lo = lo + (1 - bit) * size
            keep_lo = lo + bit * size
            partner = my_i ^ (1 << r)
            sstage_ref[pl.ds(off, size), :] = acc_ref[
                pl.ds(send_lo, size), :].astype(jnp.bfloat16)
            rdma = pltpu.make_async_remote_copy(
                src_ref=sstage_ref.at[pl.ds(off, size), :],
                dst_ref=rstage_ref.at[pl.ds(off, size), :],
                send_sem=rs_send.at[r],
                recv_sem=rs_recv.at[r],
                device_id=(partner,),
                device_id_type=pl.DeviceIdType.MESH,
            )
            rdma.start()
            if r == 0:
                @pl.when(bit0 == 0)
                def _():
                    compute_batch(0)

                @pl.when(bit0 == 1)
                def _():
                    compute_batch(1)

            rdma.wait()
            acc_ref[pl.ds(keep_lo, size), :] = (
                acc_ref[pl.ds(keep_lo, size), :]
                + rstage_ref[pl.ds(off, size), :].astype(jnp.float32))
            lo = keep_lo
            off += size

        g_ref[pl.ds(lo, CHUNK), :] = acc_ref[
            pl.ds(lo, CHUNK), :].astype(jnp.bfloat16)
        rev = lo // CHUNK

        for j in range(4):
            size = CHUNK << j
            send_lo = ((rev >> j) << j) * CHUNK
            partner = my_i ^ (8 >> j)
            rdma = pltpu.make_async_remote_copy(
                src_ref=g_ref.at[pl.ds(send_lo, size), :],
                dst_ref=g_ref.at[pl.ds(send_lo, size), :],
                send_sem=ag_send.at[j],
                recv_sem=ag_recv.at[j],
                device_id=(partner,),
                device_id_type=pl.DeviceIdType.MESH,
            )
            rdma.start()
            rdma.wait()

        out_ref[...] = g_ref[...].astype(jnp.float32)

    out = pl.pallas_call(
        body,
        out_shape=jax.ShapeDtypeStruct((ROWS, D), jnp.float32),
        in_specs=[pl.BlockSpec(memory_space=pltpu.VMEM)] * 5,
        out_specs=pl.BlockSpec(memory_space=pltpu.VMEM),
        scratch_shapes=[
            pltpu.VMEM((ROWS, HQ_LOC * DH), jnp.float32),
            pltpu.VMEM((ROWS, D), jnp.float32),
            pltpu.VMEM((480, D), jnp.bfloat16),
            pltpu.VMEM((480, D), jnp.bfloat16),
            pltpu.VMEM((ROWS, D), jnp.bfloat16),
            pltpu.SemaphoreType.DMA((4,)),
            pltpu.SemaphoreType.DMA((4,)),
            pltpu.SemaphoreType.DMA((4,)),
            pltpu.SemaphoreType.DMA((4,)),
        ],
        compiler_params=pltpu.CompilerParams(collective_id=0),
    )(x2, Wq, Wo, Wk3, Wv3)
    return out.reshape(B, SQ, D)


# device time: 46895 ns/iter; 2.2404x vs baseline; 1.0178x over previous
import jax
import jax.numpy as jnp
from jax import lax
from jax.experimental import pallas as pl
from jax.experimental.pallas import tpu as pltpu

N_DEV = 16
B, SQ, D = 2, 256, 768
DH = 64
HQ_LOC = 8
HKV_LOC = 2
ROWS = B * SQ
CHUNK = ROWS // N_DEV


def kernel(x, Wq, Wo, Wk, Wv):
    x2 = x.reshape(ROWS, D)
    Wk3 = Wk.reshape(D, N_DEV, HKV_LOC * DH)
    Wv3 = Wv.reshape(D, N_DEV, HKV_LOC * DH)

    def body(x_ref, wq_ref, wo_ref, wk_ref, wv_ref, out_ref,
             attn_ref, acc_ref, sstage_ref, rstage_ref, g_ref,
             rs_send, rs_recv, ag_send, ag_recv):
        my_i = lax.axis_index("i")

        barrier = pltpu.get_barrier_semaphore()
        for lg in range(4):
            pl.semaphore_signal(barrier, inc=1, device_id=(my_i ^ (1 << lg),),
                                device_id_type=pl.DeviceIdType.MESH)
        pl.semaphore_wait(barrier, 4)

        wq = wq_ref[...].astype(jnp.bfloat16)
        wo = wo_ref[...].astype(jnp.bfloat16)
        wk_loc = wk_ref[:, my_i, :].astype(jnp.bfloat16)
        wv_loc = wv_ref[:, my_i, :].astype(jnp.bfloat16)

        def compute_batch(b):
            r0, r1 = b * SQ, (b + 1) * SQ
            xb = x_ref[r0:r1, :].astype(jnp.bfloat16)
            q = jnp.dot(xb, wq,
                        preferred_element_type=jnp.float32).astype(jnp.bfloat16)
            k = jnp.dot(xb, wk_loc,
                        preferred_element_type=jnp.float32).astype(jnp.bfloat16)
            v = jnp.dot(xb, wv_loc,
                        preferred_element_type=jnp.float32).astype(jnp.bfloat16)
            for h in range(HQ_LOC):
                g = h // 4
                qh = q[:, h * DH:(h + 1) * DH]
                kh = k[:, g * DH:(g + 1) * DH]
                vh = v[:, g * DH:(g + 1) * DH]
                s = lax.dot_general(
                    qh, kh, (((1,), (1,)), ((), ())),
                    preferred_element_type=jnp.float32) * 0.125
                m = jnp.max(s, axis=1, keepdims=True)
                p = jnp.exp(s - m)
                l = jnp.sum(p, axis=1, keepdims=True)
                o = jnp.dot(p.astype(jnp.bfloat16), vh,
                            preferred_element_type=jnp.float32) / l
                attn_ref[r0:r1, h * DH:(h + 1) * DH] = o
            acc_ref[r0:r1, :] = jnp.dot(
                attn_ref[r0:r1, :].astype(jnp.bfloat16), wo,
                preferred_element_type=jnp.float32)

        bit0 = my_i & 1

        @pl.when(bit0 == 0)
        def _():
            compute_batch(1)

        @pl.when(bit0 == 1)
        def _():
            compute_batch(0)

        lo = 0
        off = 0
        for r in range(4):
            size = 256 >> r
            bit = (my_i >> r) & 1
            send_lo = lo + (1 - bit) * size
            keep_lo = lo + bit * size
            partner = my_i ^ (1 << r)
            sstage_ref[pl.ds(off, size), :] = acc_ref[
                pl.ds(send_lo, size), :].astype(jnp.bfloat16)
            rdma = pltpu.make_async_remote_copy(
                src_ref=sstage_ref.at[pl.ds(off, size), :],
                dst_ref=rstage_ref.at[pl.ds(off, size), :],
                send_sem=rs_send.at[r],
                recv_sem=rs_recv.at[r],
                device_id=(partner,),
                device_id_type=pl.DeviceIdType.MESH,
            )
            rdma.start()
            if r == 0:
                @pl.when(bit0 == 0)
                def _():
                    compute_batch(0)

                @pl.when(bit0 == 1)
                def _():
                    compute_batch(1)

            rdma.wait()
            acc_ref[pl.ds(keep_lo, size), :] = (
                acc_ref[pl.ds(keep_lo, size), :]
                + rstage_ref[pl.ds(off, size), :].astype(jnp.float32))
            lo = keep_lo
            off += size

        g_ref[pl.ds(lo, CHUNK), :] = acc_ref[
            pl.ds(lo, CHUNK), :].astype(jnp.bfloat16)
        rev = lo // CHUNK

        for j in range(4):
            size = CHUNK << j
            send_lo = ((rev >> j) << j) * CHUNK
            partner = my_i ^ (8 >> j)
            rdma = pltpu.make_async_remote_copy(
                src_ref=g_ref.at[pl.ds(send_lo, size), :],
                dst_ref=g_ref.at[pl.ds(send_lo, size), :],
                send_sem=ag_send.at[j],
                recv_sem=ag_recv.at[j],
                device_id=(partner,),
                device_id_type=pl.DeviceIdType.MESH,
            )
            rdma.start()
            rdma.wait()

        out_ref[...] = g_ref[...].astype(jnp.float32)

    out = pl.pallas_call(
        body,
        out_shape=jax.ShapeDtypeStruct((ROWS, D), jnp.float32),
        in_specs=[pl.BlockSpec(memory_space=pltpu.VMEM)] * 5,
        out_specs=pl.BlockSpec(memory_space=pltpu.VMEM),
        scratch_shapes=[
            pltpu.VMEM((ROWS, HQ_LOC * DH), jnp.float32),
            pltpu.VMEM((ROWS, D), jnp.float32),
            pltpu.VMEM((480, D), jnp.bfloat16),
            pltpu.VMEM((480, D), jnp.bfloat16),
            pltpu.VMEM((ROWS, D), jnp.bfloat16),
            pltpu.SemaphoreType.DMA((4,)),
            pltpu.SemaphoreType.DMA((4,)),
            pltpu.SemaphoreType.DMA((4,)),
            pltpu.SemaphoreType.DMA((4,)),
        ],
        compiler_params=pltpu.CompilerParams(collective_id=0),
    )(x2, Wq, Wo, Wk3, Wv3)
    return out.reshape(B, SQ, D)


# device time: 40065 ns/iter; 2.6224x vs baseline; 1.1705x over previous
import jax
import jax.numpy as jnp
from jax import lax
from jax.experimental import pallas as pl
from jax.experimental.pallas import tpu as pltpu

N_DEV = 16
B, SQ, D = 2, 256, 768
DH = 64
HQ_LOC = 8
HKV_LOC = 2
ROWS = B * SQ
CHUNK = ROWS // N_DEV


def kernel(x, Wq, Wo, Wk, Wv):
    x2 = x.reshape(ROWS, D)
    Wk3 = Wk.reshape(D, N_DEV, HKV_LOC * DH)
    Wv3 = Wv.reshape(D, N_DEV, HKV_LOC * DH)

    def body(x_ref, wq_ref, wo_ref, wk_ref, wv_ref, out_ref,
             attn_ref, acc_ref, sstage_ref, stage_ref, g_ref,
             rs_send, rs_recv, ag_send, ag_recv):
        my_i = lax.axis_index("i")

        barrier = pltpu.get_barrier_semaphore()
        for k in range(1, N_DEV):
            pl.semaphore_signal(barrier, inc=1,
                                device_id=((my_i + k) % N_DEV,),
                                device_id_type=pl.DeviceIdType.MESH)
        pl.semaphore_wait(barrier, N_DEV - 1)

        wq = wq_ref[...].astype(jnp.bfloat16)
        wo = wo_ref[...].astype(jnp.bfloat16)
        wk_loc = wk_ref[:, my_i, :].astype(jnp.bfloat16)
        wv_loc = wv_ref[:, my_i, :].astype(jnp.bfloat16)

        def compute_batch(b):
            r0, r1 = b * SQ, (b + 1) * SQ
            xb = x_ref[r0:r1, :].astype(jnp.bfloat16)
            q = jnp.dot(xb, wq,
                        preferred_element_type=jnp.float32).astype(jnp.bfloat16)
            k = jnp.dot(xb, wk_loc,
                        preferred_element_type=jnp.float32).astype(jnp.bfloat16)
            v = jnp.dot(xb, wv_loc,
                        preferred_element_type=jnp.float32).astype(jnp.bfloat16)
            for h in range(HQ_LOC):
                g = h // 4
                qh = q[:, h * DH:(h + 1) * DH]
                kh = k[:, g * DH:(g + 1) * DH]
                vh = v[:, g * DH:(g + 1) * DH]
                s = lax.dot_general(
                    qh, kh, (((1,), (1,)), ((), ())),
                    preferred_element_type=jnp.float32) * 0.125
                m = jnp.max(s, axis=1, keepdims=True)
                p = jnp.exp(s - m)
                l = jnp.sum(p, axis=1, keepdims=True)
                o = jnp.dot(p.astype(jnp.bfloat16), vh,
                            preferred_element_type=jnp.float32) / l
                attn_ref[r0:r1, h * DH:(h + 1) * DH] = o
            acc_ref[r0:r1, :] = jnp.dot(
                attn_ref[r0:r1, :].astype(jnp.bfloat16), wo,
                preferred_element_type=jnp.float32)

        def rs_rdma(k):
            dest = (my_i + k) % N_DEV
            return pltpu.make_async_remote_copy(
                src_ref=sstage_ref.at[pl.ds(dest * CHUNK, CHUNK), :],
                dst_ref=stage_ref.at[pl.ds(my_i * CHUNK, CHUNK), :],
                send_sem=rs_send.at[k],
                recv_sem=rs_recv.at[k],
                device_id=(dest,),
                device_id_type=pl.DeviceIdType.MESH,
            )

        compute_batch(1)
        sstage_ref[SQ:, :] = acc_ref[SQ:, :].astype(jnp.bfloat16)
        for k in range(1, N_DEV):
            dest = (my_i + k) % N_DEV

            @pl.when(dest >= N_DEV // 2)
            def _():
                rs_rdma(k).start()

        compute_batch(0)
        sstage_ref[:SQ, :] = acc_ref[:SQ, :].astype(jnp.bfloat16)
        for k in range(1, N_DEV):
            dest = (my_i + k) % N_DEV

            @pl.when(dest < N_DEV // 2)
            def _():
                rs_rdma(k).start()

        stage_ref[pl.ds(my_i * CHUNK, CHUNK), :] = acc_ref[
            pl.ds(my_i * CHUNK, CHUNK), :].astype(jnp.bfloat16)
        for k in range(1, N_DEV):
            rs_rdma(k).wait_recv()
        red = stage_ref[0:CHUNK, :].astype(jnp.float32)
        for c in range(1, N_DEV):
            red = red + stage_ref[c * CHUNK:(c + 1) * CHUNK, :].astype(
                jnp.float32)

        g_ref[pl.ds(my_i * CHUNK, CHUNK), :] = red.astype(jnp.bfloat16)

        def ag_rdma(k):
            return pltpu.make_async_remote_copy(
                src_ref=g_ref.at[pl.ds(my_i * CHUNK, CHUNK), :],
                dst_ref=g_ref.at[pl.ds(my_i * CHUNK, CHUNK), :],
                send_sem=ag_send.at[k],
                recv_sem=ag_recv.at[k],
                device_id=((my_i + k) % N_DEV,),
                device_id_type=pl.DeviceIdType.MESH,
            )

        for k in range(1, N_DEV):
            ag_rdma(k).start()
        for k in range(1, N_DEV):
            ag_rdma(k).wait_recv()
        out_ref[...] = g_ref[...].astype(jnp.float32)

        for k in range(1, N_DEV):
            rs_rdma(k).wait_send()
            ag_rdma(k).wait_send()

    out = pl.pallas_call(
        body,
        out_shape=jax.ShapeDtypeStruct((ROWS, D), jnp.float32),
        in_specs=[pl.BlockSpec(memory_space=pltpu.VMEM)] * 5,
        out_specs=pl.BlockSpec(memory_space=pltpu.VMEM),
        scratch_shapes=[
            pltpu.VMEM((ROWS, HQ_LOC * DH), jnp.float32),
            pltpu.VMEM((ROWS, D), jnp.float32),
            pltpu.VMEM((ROWS, D), jnp.bfloat16),
            pltpu.VMEM((ROWS, D), jnp.bfloat16),
            pltpu.VMEM((ROWS, D), jnp.bfloat16),
            pltpu.SemaphoreType.DMA((N_DEV,)),
            pltpu.SemaphoreType.DMA((N_DEV,)),
            pltpu.SemaphoreType.DMA((N_DEV,)),
            pltpu.SemaphoreType.DMA((N_DEV,)),
        ],
        compiler_params=pltpu.CompilerParams(collective_id=0),
    )(x2, Wq, Wo, Wk3, Wv3)
    return out.reshape(B, SQ, D)


# device time: 37765 ns/iter; 2.7821x vs baseline; 1.0609x over previous
import jax
import jax.numpy as jnp
from jax import lax
from jax.experimental import pallas as pl
from jax.experimental.pallas import tpu as pltpu

N_DEV = 16
B, SQ, D = 2, 256, 768
DH = 64
HQ_LOC = 8
HKV_LOC = 2
ROWS = B * SQ
CHUNK = ROWS // N_DEV


def kernel(x, Wq, Wo, Wk, Wv):
    x2 = x.reshape(ROWS, D)
    Wk3 = Wk.reshape(D, N_DEV, HKV_LOC * DH)
    Wv3 = Wv.reshape(D, N_DEV, HKV_LOC * DH)

    def body(x_ref, wq_ref, wo_ref, wk_ref, wv_ref, out_ref,
             attn_ref, acc_ref, sstage_ref, stage_ref, g_ref,
             rs_send, rs_recv, ag_send, ag_recv):
        my_i = lax.axis_index("i")

        barrier = pltpu.get_barrier_semaphore()
        for k in range(1, N_DEV):
            pl.semaphore_signal(barrier, inc=1,
                                device_id=((my_i + k) % N_DEV,),
                                device_id_type=pl.DeviceIdType.MESH)
        pl.semaphore_wait(barrier, N_DEV - 1)

        wq = wq_ref[...].astype(jnp.bfloat16)
        wo = wo_ref[...].astype(jnp.bfloat16)
        wk_loc = wk_ref[:, my_i, :].astype(jnp.bfloat16)
        wv_loc = wv_ref[:, my_i, :].astype(jnp.bfloat16)

        def compute_batch(b):
            r0, r1 = b * SQ, (b + 1) * SQ
            xb = x_ref[r0:r1, :].astype(jnp.bfloat16)
            q = jnp.dot(xb, wq,
                        preferred_element_type=jnp.float32).astype(jnp.bfloat16)
            k = jnp.dot(xb, wk_loc,
                        preferred_element_type=jnp.float32).astype(jnp.bfloat16)
            v = jnp.dot(xb, wv_loc,
                        preferred_element_type=jnp.float32).astype(jnp.bfloat16)
            for h in range(HQ_LOC):
                g = h // 4
                qh = q[:, h * DH:(h + 1) * DH]
                kh = k[:, g * DH:(g + 1) * DH]
                vh = v[:, g * DH:(g + 1) * DH]
                s = lax.dot_general(
                    qh, kh, (((1,), (1,)), ((), ())),
                    preferred_element_type=jnp.float32) * 0.125
                m = jnp.max(s, axis=1, keepdims=True)
                p = jnp.exp(s - m)
                l = jnp.sum(p, axis=1, keepdims=True)
                o = jnp.dot(p.astype(jnp.bfloat16), vh,
                            preferred_element_type=jnp.float32) / l
                attn_ref[r0:r1, h * DH:(h + 1) * DH] = o
            acc_ref[r0:r1, :] = jnp.dot(
                attn_ref[r0:r1, :].astype(jnp.bfloat16), wo,
                preferred_element_type=jnp.float32)

        def rs_rdma(k):
            dest = (my_i + k) % N_DEV
            return pltpu.make_async_remote_copy(
                src_ref=sstage_ref.at[pl.ds(dest * CHUNK, CHUNK), :],
                dst_ref=stage_ref.at[pl.ds(my_i * CHUNK, CHUNK), :],
                send_sem=rs_send.at[k],
                recv_sem=rs_recv.at[k],
                device_id=(dest,),
                device_id_type=pl.DeviceIdType.MESH,
            )

        hi_first = (my_i & 1) == 0

        def compute_and_send(batch):
            r0 = batch * SQ
            sstage_ref[r0:r0 + SQ, :] = acc_ref[
                r0:r0 + SQ, :].astype(jnp.bfloat16)

        @pl.when(hi_first)
        def _():
            compute_batch(1)
            compute_and_send(1)

        @pl.when(jnp.logical_not(hi_first))
        def _():
            compute_batch(0)
            compute_and_send(0)

        for k in range(1, N_DEV):
            dest = (my_i + k) % N_DEV

            @pl.when((dest >= N_DEV // 2) == hi_first)
            def _():
                rs_rdma(k).start()

        @pl.when(hi_first)
        def _():
            compute_batch(0)
            compute_and_send(0)

        @pl.when(jnp.logical_not(hi_first))
        def _():
            compute_batch(1)
            compute_and_send(1)

        for k in range(1, N_DEV):
            dest = (my_i + k) % N_DEV

            @pl.when((dest < N_DEV // 2) == hi_first)
            def _():
                rs_rdma(k).start()

        stage_ref[pl.ds(my_i * CHUNK, CHUNK), :] = acc_ref[
            pl.ds(my_i * CHUNK, CHUNK), :].astype(jnp.bfloat16)
        for k in range(1, N_DEV):
            rs_rdma(k).wait_recv()
        red = stage_ref[0:CHUNK, :].astype(jnp.float32)
        for c in range(1, N_DEV):
            red = red + stage_ref[c * CHUNK:(c + 1) * CHUNK, :].astype(
                jnp.float32)

        g_ref[pl.ds(my_i * CHUNK, CHUNK), :] = red.astype(jnp.bfloat16)

        def ag_rdma(k):
            return pltpu.make_async_remote_copy(
                src_ref=g_ref.at[pl.ds(my_i * CHUNK, CHUNK), :],
                dst_ref=g_ref.at[pl.ds(my_i * CHUNK, CHUNK), :],
                send_sem=ag_send.at[k],
                recv_sem=ag_recv.at[k],
                device_id=((my_i + k) % N_DEV,),
                device_id_type=pl.DeviceIdType.MESH,
            )

        for k in range(1, N_DEV):
            ag_rdma(k).start()
        for k in range(1, N_DEV):
            ag_rdma(k).wait_recv()
        out_ref[...] = g_ref[...].astype(jnp.float32)

        for k in range(1, N_DEV):
            rs_rdma(k).wait_send()
            ag_rdma(k).wait_send()

    out = pl.pallas_call(
        body,
        out_shape=jax.ShapeDtypeStruct((ROWS, D), jnp.float32),
        in_specs=[pl.BlockSpec(memory_space=pltpu.VMEM)] * 5,
        out_specs=pl.BlockSpec(memory_space=pltpu.VMEM),
        scratch_shapes=[
            pltpu.VMEM((ROWS, HQ_LOC * DH), jnp.float32),
            pltpu.VMEM((ROWS, D), jnp.float32),
            pltpu.VMEM((ROWS, D), jnp.bfloat16),
            pltpu.VMEM((ROWS, D), jnp.bfloat16),
            pltpu.VMEM((ROWS, D), jnp.bfloat16),
            pltpu.SemaphoreType.DMA((N_DEV,)),
            pltpu.SemaphoreType.DMA((N_DEV,)),
            pltpu.SemaphoreType.DMA((N_DEV,)),
            pltpu.SemaphoreType.DMA((N_DEV,)),
        ],
        compiler_params=pltpu.CompilerParams(collective_id=0),
    )(x2, Wq, Wo, Wk3, Wv3)
    return out.reshape(B, SQ, D)


# device time: 37689 ns/iter; 2.7877x vs baseline; 1.0020x over previous
import jax
import jax.numpy as jnp
from jax import lax
from jax.experimental import pallas as pl
from jax.experimental.pallas import tpu as pltpu

N_DEV = 16
B, SQ, D = 2, 256, 768
DH = 64
HQ_LOC = 8
HKV_LOC = 2
ROWS = B * SQ
CHUNK = ROWS // N_DEV


def kernel(x, Wq, Wo, Wk, Wv):
    x2 = x.reshape(ROWS, D)
    Wk3 = Wk.reshape(D, N_DEV, HKV_LOC * DH)
    Wv3 = Wv.reshape(D, N_DEV, HKV_LOC * DH)

    def body(x_ref, wq_ref, wo_ref, wk_ref, wv_ref, out_ref,
             attn_ref, acc_ref, sstage_ref, stage_ref, g_ref,
             rs_send, rs_recv, ag_send, ag_recv):
        my_i = lax.axis_index("i")

        wq = wq_ref[...].astype(jnp.bfloat16)
        wo = wo_ref[...].astype(jnp.bfloat16)
        wk_loc = wk_ref[:, my_i, :].astype(jnp.bfloat16)
        wv_loc = wv_ref[:, my_i, :].astype(jnp.bfloat16)

        def compute_batch(b):
            r0, r1 = b * SQ, (b + 1) * SQ
            xb = x_ref[r0:r1, :].astype(jnp.bfloat16)
            q = jnp.dot(xb, wq,
                        preferred_element_type=jnp.float32).astype(jnp.bfloat16)
            k = jnp.dot(xb, wk_loc,
                        preferred_element_type=jnp.float32).astype(jnp.bfloat16)
            v = jnp.dot(xb, wv_loc,
                        preferred_element_type=jnp.float32).astype(jnp.bfloat16)
            for h in range(HQ_LOC):
                g = h // 4
                qh = q[:, h * DH:(h + 1) * DH]
                kh = k[:, g * DH:(g + 1) * DH]
                vh = v[:, g * DH:(g + 1) * DH]
                s = lax.dot_general(
                    qh, kh, (((1,), (1,)), ((), ())),
                    preferred_element_type=jnp.float32) * 0.125
                m = jnp.max(s, axis=1, keepdims=True)
                p = jnp.exp(s - m)
                l = jnp.sum(p, axis=1, keepdims=True)
                o = jnp.dot(p.astype(jnp.bfloat16), vh,
                            preferred_element_type=jnp.float32) / l
                attn_ref[r0:r1, h * DH:(h + 1) * DH] = o
            acc_ref[r0:r1, :] = jnp.dot(
                attn_ref[r0:r1, :].astype(jnp.bfloat16), wo,
                preferred_element_type=jnp.float32)

        def rs_rdma(k):
            dest = (my_i + k) % N_DEV
            return pltpu.make_async_remote_copy(
                src_ref=sstage_ref.at[pl.ds(dest * CHUNK, CHUNK), :],
                dst_ref=stage_ref.at[pl.ds(my_i * CHUNK, CHUNK), :],
                send_sem=rs_send.at[k],
                recv_sem=rs_recv.at[k],
                device_id=(dest,),
                device_id_type=pl.DeviceIdType.MESH,
            )

        hi_first = (my_i & 1) == 0

        def compute_and_send(batch):
            r0 = batch * SQ
            sstage_ref[r0:r0 + SQ, :] = acc_ref[
                r0:r0 + SQ, :].astype(jnp.bfloat16)

        @pl.when(hi_first)
        def _():
            compute_batch(1)
            compute_and_send(1)

        @pl.when(jnp.logical_not(hi_first))
        def _():
            compute_batch(0)
            compute_and_send(0)

        barrier = pltpu.get_barrier_semaphore()
        for k in range(1, N_DEV):
            pl.semaphore_signal(barrier, inc=1,
                                device_id=((my_i + k) % N_DEV,),
                                device_id_type=pl.DeviceIdType.MESH)
        pl.semaphore_wait(barrier, N_DEV - 1)

        for k in range(1, N_DEV):
            dest = (my_i + k) % N_DEV

            @pl.when((dest >= N_DEV // 2) == hi_first)
            def _():
                rs_rdma(k).start()

        @pl.when(hi_first)
        def _():
            compute_batch(0)
            compute_and_send(0)

        @pl.when(jnp.logical_not(hi_first))
        def _():
            compute_batch(1)
            compute_and_send(1)

        for k in range(1, N_DEV):
            dest = (my_i + k) % N_DEV

            @pl.when((dest < N_DEV // 2) == hi_first)
            def _():
                rs_rdma(k).start()

        red = acc_ref[pl.ds(my_i * CHUNK, CHUNK), :]
        for k in range(1, N_DEV):
            rs_rdma(k).wait_recv()
            sender = (my_i + N_DEV - k) % N_DEV
            red = red + stage_ref[pl.ds(sender * CHUNK, CHUNK), :].astype(
                jnp.float32)

        g_ref[pl.ds(my_i * CHUNK, CHUNK), :] = red.astype(jnp.bfloat16)

        def ag_rdma(k):
            return pltpu.make_async_remote_copy(
                src_ref=g_ref.at[pl.ds(my_i * CHUNK, CHUNK), :],
                dst_ref=g_ref.at[pl.ds(my_i * CHUNK, CHUNK), :],
                send_sem=ag_send.at[k],
                recv_sem=ag_recv.at[k],
                device_id=((my_i + k) % N_DEV,),
                device_id_type=pl.DeviceIdType.MESH,
            )

        for k in range(1, N_DEV):
            ag_rdma(k).start()
        for k in range(1, N_DEV):
            ag_rdma(k).wait_recv()
        out_ref[...] = g_ref[...].astype(jnp.float32)

        for k in range(1, N_DEV):
            rs_rdma(k).wait_send()
            ag_rdma(k).wait_send()

    out = pl.pallas_call(
        body,
        out_shape=jax.ShapeDtypeStruct((ROWS, D), jnp.float32),
        in_specs=[pl.BlockSpec(memory_space=pltpu.VMEM)] * 5,
        out_specs=pl.BlockSpec(memory_space=pltpu.VMEM),
        scratch_shapes=[
            pltpu.VMEM((ROWS, HQ_LOC * DH), jnp.float32),
            pltpu.VMEM((ROWS, D), jnp.float32),
            pltpu.VMEM((ROWS, D), jnp.bfloat16),
            pltpu.VMEM((ROWS, D), jnp.bfloat16),
            pltpu.VMEM((ROWS, D), jnp.bfloat16),
            pltpu.SemaphoreType.DMA((N_DEV,)),
            pltpu.SemaphoreType.DMA((N_DEV,)),
            pltpu.SemaphoreType.DMA((N_DEV,)),
            pltpu.SemaphoreType.DMA((N_DEV,)),
        ],
        compiler_params=pltpu.CompilerParams(collective_id=0),
    )(x2, Wq, Wo, Wk3, Wv3)
    return out.reshape(B, SQ, D)
